# baseline (speedup 1.0000x reference)
import functools

import jax
import jax.numpy as jnp
from jax import lax
from jax.experimental import pallas as pl
from jax.experimental.pallas import tpu as pltpu
from jax.experimental.pallas import tpu_sc as plsc

VOCAB = 1000000
EMBED = 64
HIDDEN = 256
OUT = 10
SEQ = 200
BATCH = 4096

NUM_CORES = 2
NUM_SUBCORES = 16
NW = NUM_CORES * NUM_SUBCORES
B_PER_W = BATCH // NW
LANES = 16
EMB_VECS = EMBED // LANES


def _pool_kernel_body(text_hbm, emb_hbm, out_hbm, idx_v, acc_v, sem):
    wid = lax.axis_index("s") * NUM_CORES + lax.axis_index("c")
    base = wid * B_PER_W

    pltpu.sync_copy(text_hbm.at[:, wid], idx_v)

    zero = jnp.zeros((LANES,), jnp.float32)

    @pl.loop(0, B_PER_W)
    def _(i):
        for c in range(EMB_VECS):
            acc_v[i, pl.ds(c * LANES, LANES)] = zero

    @pl.loop(0, SEQ)
    def _(s):
        pltpu.async_copy(emb_hbm.at[idx_v.at[s]], acc_v, sem, add=True)

    @pl.loop(0, SEQ)
    def _(s):
        pltpu.make_async_copy(emb_hbm.at[idx_v.at[s]], acc_v, sem).wait()

    pltpu.sync_copy(acc_v, out_hbm.at[pl.ds(base, B_PER_W)])


@functools.partial(
    pl.kernel,
    out_type=jax.ShapeDtypeStruct((BATCH, EMBED), jnp.float32),
    mesh=plsc.VectorSubcoreMesh(core_axis_name="c", subcore_axis_name="s"),
    compiler_params=pltpu.CompilerParams(use_tc_tiling_on_sc=False),
    scratch_types=[
        pltpu.VMEM((SEQ, B_PER_W), jnp.int32),
        pltpu.VMEM((B_PER_W, EMBED), jnp.float32),
        pltpu.SemaphoreType.DMA,
    ],
)
def _pool_kernel(text_hbm, emb_hbm, out_hbm, idx_v, acc_v, sem):
    _pool_kernel_body(text_hbm, emb_hbm, out_hbm, idx_v, acc_v, sem)


def _mlp_body(pooled_ref, w1_ref, b1_ref, w2_ref, b2_ref, out_ref):
    pooled = pooled_ref[...] * jnp.float32(1.0 / SEQ)
    hidden = (
        jnp.dot(pooled, w1_ref[...], preferred_element_type=jnp.float32)
        + b1_ref[...])
    out_ref[...] = (
        jnp.dot(hidden, w2_ref[...], preferred_element_type=jnp.float32)
        + b2_ref[...])


def _mlp(pooled, W1, b1, W2, b2):
    return pl.pallas_call(
        _mlp_body,
        out_shape=jax.ShapeDtypeStruct((BATCH, OUT), jnp.float32),
    )(pooled, W1, b1.reshape(1, HIDDEN), W2, b2.reshape(1, OUT))


@jax.jit
def kernel(text, emb, W1, b1, W2, b2):
    if text.dtype != jnp.int32:
        text = text.astype(jnp.int32)
    t3 = text.reshape(SEQ, NW, B_PER_W)
    pooled = _pool_kernel(t3, emb)
    return _mlp(pooled, W1, b1, W2, b2)


# speedup vs baseline: 1.1050x; 1.1050x over previous
import functools

import jax
import jax.numpy as jnp
from jax import lax
from jax.experimental import pallas as pl
from jax.experimental.pallas import tpu as pltpu
from jax.experimental.pallas import tpu_sc as plsc

VOCAB = 1000000
EMBED = 64
HIDDEN = 256
OUT = 10
SEQ = 200
BATCH = 4096

NUM_CORES = 2
NUM_SUBCORES = 16
NW = NUM_CORES * NUM_SUBCORES
B_PER_W = BATCH // NW
LANES = 16
EMB_VECS = EMBED // LANES


def _pool_kernel_body(text_hbm, emb_hbm, out_hbm, idx_v, acc_v, sem):
    wid = lax.axis_index("s") * NUM_CORES + lax.axis_index("c")
    base = wid * B_PER_W

    pltpu.sync_copy(text_hbm.at[:, wid], idx_v)

    zero = jnp.zeros((LANES,), jnp.float32)

    @pl.loop(0, B_PER_W)
    def _(i):
        for c in range(EMB_VECS):
            acc_v[i, pl.ds(c * LANES, LANES)] = zero

    @pl.loop(0, SEQ)
    def _(s):
        pltpu.async_copy(emb_hbm.at[idx_v.at[s]], acc_v, sem, add=True)

    @pl.loop(0, SEQ)
    def _(s):
        pltpu.make_async_copy(emb_hbm.at[idx_v.at[s]], acc_v, sem).wait()

    pltpu.sync_copy(acc_v, out_hbm.at[pl.ds(base, B_PER_W)])


@functools.partial(
    pl.kernel,
    out_type=jax.ShapeDtypeStruct((BATCH, EMBED), jnp.float32),
    mesh=plsc.VectorSubcoreMesh(core_axis_name="c", subcore_axis_name="s"),
    compiler_params=pltpu.CompilerParams(use_tc_tiling_on_sc=False),
    scratch_types=[
        pltpu.VMEM((SEQ, B_PER_W), jnp.int32),
        pltpu.VMEM((B_PER_W, EMBED), jnp.float32),
        pltpu.SemaphoreType.DMA,
    ],
)
def _pool_kernel(text_hbm, emb_hbm, out_hbm, idx_v, acc_v, sem):
    _pool_kernel_body(text_hbm, emb_hbm, out_hbm, idx_v, acc_v, sem)


def _mlp_body(pooled_ref, w1_ref, b1_ref, w2_ref, b2_ref, out_ref):
    pooled = pooled_ref[...] * jnp.float32(1.0 / SEQ)
    hidden = (
        jnp.dot(pooled, w1_ref[...], preferred_element_type=jnp.float32)
        + b1_ref[...])
    out_ref[...] = (
        jnp.dot(hidden, w2_ref[...], preferred_element_type=jnp.float32)
        + b2_ref[...])


def _mlp(pooled, W1, b1, W2, b2):
    return pl.pallas_call(
        _mlp_body,
        out_shape=jax.ShapeDtypeStruct((BATCH, OUT), jnp.float32),
    )(pooled, W1, b1.reshape(1, HIDDEN), W2, b2.reshape(1, OUT))


@jax.jit
def kernel(text, emb, W1, b1, W2, b2):
    if text.dtype != jnp.int32:
        text = text.astype(jnp.int32)
    t3 = (text * 2).reshape(SEQ, NW, B_PER_W)
    emb2 = jnp.pad(emb, ((0, 0), (0, EMBED))).reshape(2 * VOCAB, EMBED)
    pooled = _pool_kernel(t3, emb2)
    return _mlp(pooled, W1, b1, W2, b2)
